# baseline (speedup 1.0000x reference)
import functools

import jax
import jax.numpy as jnp
from jax import lax
from jax.experimental import pallas as pl
from jax.experimental.pallas import tpu as pltpu
from jax.experimental.pallas import tpu_sc as plsc

EMBED_DIM = 32
PAD_ROW = EMBED_DIM + 1
NUM_CORES = 2
NUM_SUBCORES = 16
NUM_WORKERS = NUM_CORES * NUM_SUBCORES
LANES = 16
SUBLANES = 8
LANES128 = 128


@functools.lru_cache(maxsize=None)
def _build(batch: int, hist: int, vocab_rows: int):
    b_per_w = batch // NUM_WORKERS
    bt_per_w = b_per_w // LANES128
    groups = b_per_w // LANES
    d_tiles = EMBED_DIM // SUBLANES
    assert batch % (NUM_WORKERS * LANES128) == 0
    mesh = plsc.VectorSubcoreMesh(core_axis_name="c", subcore_axis_name="s")

    @functools.partial(
        pl.kernel,
        mesh=mesh,
        compiler_params=pltpu.CompilerParams(
            use_tc_tiling_on_sc=False, needs_layout_passes=False),
        out_type=jax.ShapeDtypeStruct(
            (hist, d_tiles, batch // LANES128, SUBLANES, LANES128),
            jnp.float32),
        scratch_types=[
            pltpu.VMEM((vocab_rows * PAD_ROW,), jnp.float32),
            pltpu.VMEM((hist, b_per_w), jnp.int32),
            pltpu.VMEM((2, d_tiles, bt_per_w, SUBLANES, LANES128),
                       jnp.float32),
            pltpu.SemaphoreType.DMA((2,)),
        ],
    )
    def gather_kernel(idxt_hbm, table_hbm, out_hbm, table_v, islab, stg,
                      osem):
        wid = lax.axis_index("s") * NUM_CORES + lax.axis_index("c")
        b0 = wid * b_per_w
        bt0 = wid * bt_per_w

        pltpu.sync_copy(table_hbm, table_v)
        pltpu.sync_copy(idxt_hbm.at[:, pl.ds(b0, b_per_w)], islab)

        def step_body(h, carry):
            buf = lax.rem(h, 2)

            @pl.when(h >= 2)
            def _drain():
                pltpu.make_async_copy(
                    stg.at[buf], out_hbm.at[h - 2, :, pl.ds(bt0, bt_per_w)],
                    osem.at[buf]).wait()

            @plsc.parallel_loop(0, groups, unroll=2)
            def group_body(g):
                vec = islab[h, pl.ds(g * LANES, LANES)]
                addr = (vec + 1) * PAD_ROW
                bt = g // (LANES128 // LANES)
                bs0 = lax.rem(g, LANES128 // LANES) * LANES
                for dt in range(d_tiles):
                    for ds in range(SUBLANES):
                        col = plsc.load_gather(
                            table_v, [addr + (dt * SUBLANES + ds)])
                        stg[buf, dt, bt, ds, pl.ds(bs0, LANES)] = col

            pltpu.async_copy(
                stg.at[buf], out_hbm.at[h, :, pl.ds(bt0, bt_per_w)],
                osem.at[buf])
            return carry

        lax.fori_loop(0, hist, step_body, 0)

        for h in (hist - 2, hist - 1):
            pltpu.make_async_copy(
                stg.at[h % 2], out_hbm.at[h, :, pl.ds(bt0, bt_per_w)],
                osem.at[h % 2]).wait()

    return gather_kernel


def kernel(indices, table):
    batch, hist = indices.shape
    idx_t = indices.T
    table_pad = jnp.pad(table, ((0, 0), (0, PAD_ROW - EMBED_DIM)))
    table_flat = table_pad.reshape(table.shape[0] * PAD_ROW)
    out5 = _build(batch, hist, table.shape[0])(idx_t, table_flat)
    return out5.transpose((2, 4, 0, 1, 3)).reshape(batch, hist, EMBED_DIM)


# speedup vs baseline: 1.2478x; 1.2478x over previous
import functools

import jax
import jax.numpy as jnp
from jax import lax
from jax.experimental import pallas as pl
from jax.experimental.pallas import tpu as pltpu
from jax.experimental.pallas import tpu_sc as plsc

EMBED_DIM = 32
PAD_ROW = EMBED_DIM + 1
NUM_CORES = 2
NUM_SUBCORES = 16
NUM_WORKERS = NUM_CORES * NUM_SUBCORES
LANES = 16
SUBLANES = 8
LANES128 = 128


@functools.lru_cache(maxsize=None)
def _build(batch: int, hist: int, vocab_rows: int):
    b_per_w = batch // NUM_WORKERS
    bt_per_w = b_per_w // LANES128
    groups = b_per_w // LANES
    d_tiles = EMBED_DIM // SUBLANES
    assert batch % (NUM_WORKERS * LANES128) == 0
    mesh = plsc.VectorSubcoreMesh(core_axis_name="c", subcore_axis_name="s")

    @functools.partial(
        pl.kernel,
        mesh=mesh,
        compiler_params=pltpu.CompilerParams(
            use_tc_tiling_on_sc=False, needs_layout_passes=False),
        out_type=jax.ShapeDtypeStruct(
            (hist, d_tiles, batch // LANES128, SUBLANES, LANES128),
            jnp.float32),
        scratch_types=[
            pltpu.VMEM((vocab_rows * PAD_ROW,), jnp.float32),
            pltpu.VMEM((hist, b_per_w), jnp.int32),
            pltpu.VMEM((2, d_tiles, bt_per_w, SUBLANES, LANES128),
                       jnp.float32),
            pltpu.SemaphoreType.DMA((2,)),
        ],
    )
    def gather_kernel(idxt_hbm, table_hbm, out_hbm, table_v, islab, stg,
                      osem):
        wid = lax.axis_index("s") * NUM_CORES + lax.axis_index("c")
        b0 = wid * b_per_w
        bt0 = wid * bt_per_w

        pltpu.sync_copy(table_hbm, table_v)
        pltpu.sync_copy(idxt_hbm.at[:, pl.ds(b0, b_per_w)], islab)

        def step_body(h, carry):
            buf = lax.rem(h, 2)

            @pl.when(h >= 2)
            def _drain():
                pltpu.make_async_copy(
                    stg.at[buf], out_hbm.at[h - 2, :, pl.ds(bt0, bt_per_w)],
                    osem.at[buf]).wait()

            @plsc.parallel_loop(0, groups, unroll=1)
            def group_body(g):
                vec = islab[h, pl.ds(g * LANES, LANES)]
                addr = (vec + 1) * PAD_ROW
                bt = g // (LANES128 // LANES)
                bs0 = lax.rem(g, LANES128 // LANES) * LANES
                for dt in range(d_tiles):
                    for ds in range(SUBLANES):
                        col = plsc.load_gather(
                            table_v, [addr + (dt * SUBLANES + ds)])
                        stg[buf, dt, bt, ds, pl.ds(bs0, LANES)] = col

            pltpu.async_copy(
                stg.at[buf], out_hbm.at[h, :, pl.ds(bt0, bt_per_w)],
                osem.at[buf])
            return carry

        lax.fori_loop(0, hist, step_body, 0)

        for h in (hist - 2, hist - 1):
            pltpu.make_async_copy(
                stg.at[h % 2], out_hbm.at[h, :, pl.ds(bt0, bt_per_w)],
                osem.at[h % 2]).wait()

    return gather_kernel


def kernel(indices, table):
    batch, hist = indices.shape
    idx_t = indices.T
    table_pad = jnp.pad(table, ((0, 0), (0, PAD_ROW - EMBED_DIM)))
    table_flat = table_pad.reshape(table.shape[0] * PAD_ROW)
    out5 = _build(batch, hist, table.shape[0])(idx_t, table_flat)
    return out5.transpose((2, 4, 0, 1, 3)).reshape(batch, hist, EMBED_DIM)
